# baseline (speedup 1.0000x reference)
import dataclasses
import functools

import jax
import jax.numpy as jnp
from jax import lax
from jax.experimental import pallas as pl
from jax.experimental.pallas import tpu as pltpu
from jax.experimental.pallas import tpu_sc as plsc

N_WORKERS = 32
LANES = 16
CHUNK = 128


def _sc_compiler_params():
    cp = pltpu.CompilerParams()
    if "needs_layout_passes" in pltpu.CompilerParams.__dataclass_fields__:
        cp = dataclasses.replace(cp, needs_layout_passes=False)
    return cp


def _sc_degree(dst, n_bins):
    n_nodes = n_bins
    n_edges = dst.shape[0]
    per_w = n_edges // N_WORKERS
    assert per_w * N_WORKERS == n_edges and per_w % LANES == 0
    mesh = plsc.VectorSubcoreMesh(core_axis_name="c", subcore_axis_name="s")

    @functools.partial(
        pl.kernel,
        out_type=jax.ShapeDtypeStruct((N_WORKERS, n_nodes), jnp.float32),
        mesh=mesh,
        scratch_types=[
            pltpu.VMEM((per_w,), jnp.int32),
            pltpu.VMEM((n_nodes,), jnp.float32),
        ],
        compiler_params=_sc_compiler_params(),
    )
    def k(dst_hbm, out_hbm, idx_v, hist_v):
        wid = lax.axis_index("s") * 2 + lax.axis_index("c")
        zero16 = jnp.zeros((LANES,), jnp.float32)
        ones16 = jnp.ones((LANES,), jnp.float32)

        @pl.loop(0, n_nodes, step=LANES)
        def _(i):
            hist_v[pl.ds(i, LANES)] = zero16

        pltpu.sync_copy(dst_hbm.at[pl.ds(wid * per_w, per_w)], idx_v)

        @pl.loop(0, per_w, step=LANES)
        def _(i):
            idx = idx_v[pl.ds(i, LANES)]
            plsc.addupdate_scatter(hist_v, [idx], ones16)

        pltpu.sync_copy(hist_v, out_hbm.at[wid])

    return k(dst)


def _tc_dinv(hist):
    n = hist.shape[1]

    def body(h_ref, o_ref):
        deg = jnp.sum(h_ref[...], axis=0) + 1.0
        o_ref[...] = lax.rsqrt(deg)

    return pl.pallas_call(
        body, out_shape=jax.ShapeDtypeStruct((n,), jnp.float32)
    )(hist)


def _tc_scale(x, dinv_col):
    def body(x_ref, d_ref, o_ref):
        o_ref[...] = x_ref[...] * d_ref[...]

    return pl.pallas_call(
        body, out_shape=jax.ShapeDtypeStruct(x.shape, x.dtype)
    )(x, dinv_col)


NSLOT = 2
ISLOT = 4


def _sc_aggregate(xs, src2d, dst2d, npad):
    n, d = xs.shape
    total_chunks = src2d.shape[0]
    cpt = total_chunks // N_WORKERS
    assert cpt * N_WORKERS == total_chunks and cpt % ISLOT == 0
    groups = cpt // ISLOT
    rows_per_tile = npad // 16
    zr = 32
    assert rows_per_tile % zr == 0
    mesh = plsc.VectorSubcoreMesh(core_axis_name="c", subcore_axis_name="s")

    @functools.partial(
        pl.kernel,
        out_type=jax.ShapeDtypeStruct((2 * npad, d), jnp.float32),
        mesh=mesh,
        scratch_types=[
            pltpu.VMEM_SHARED((npad, d), jnp.float32),
            pltpu.VMEM((zr, d), jnp.float32),
            pltpu.VMEM((ISLOT, CHUNK), jnp.int32),
            pltpu.VMEM((ISLOT, CHUNK), jnp.int32),
            pltpu.VMEM((NSLOT, CHUNK, d), jnp.float32),
            pltpu.SemaphoreType.DMA((ISLOT,)),
            pltpu.SemaphoreType.DMA((NSLOT,)),
        ],
    )
    def k(xs_hbm, src_hbm, dst_hbm, out_hbm, t_sh, zbuf, sidx, didx, rows,
          isem, gsem):
        cid = lax.axis_index("c")
        sid = lax.axis_index("s")
        wid = cid * 16 + sid
        crow0 = wid * cpt
        zero16 = jnp.zeros((LANES,), jnp.float32)

        @pl.loop(0, zr)
        def _(r):
            for j in range(d // LANES):
                zbuf[r, pl.ds(j * LANES, LANES)] = zero16

        row0 = sid * rows_per_tile
        for j in range(rows_per_tile // zr):
            pltpu.sync_copy(zbuf, t_sh.at[pl.ds(row0 + j * zr, zr)])
        plsc.subcore_barrier()

        def idx_descs(c, s):
            return (
                pltpu.make_async_copy(src_hbm.at[crow0 + c], sidx.at[s],
                                      isem.at[s]),
                pltpu.make_async_copy(dst_hbm.at[crow0 + c], didx.at[s],
                                      isem.at[s]),
            )

        def gather(s, b):
            return pltpu.make_async_copy(
                xs_hbm.at[sidx.at[s]], rows.at[b], gsem.at[b])

        for c0 in range(2):
            for dsc in idx_descs(c0, c0):
                dsc.start()
        for dsc in idx_descs(0, 0):
            dsc.wait()
        gather(0, 0).start()

        @pl.loop(0, groups)
        def _(g):
            for b in range(ISLOT):
                c = g * ISLOT + b

                def prefetch(c=c, s=(b + 2) % ISLOT):
                    for dsc in idx_descs(c + 2, s):
                        dsc.start()

                def launch(c=c, s=(b + 1) % ISLOT, r=(b + 1) % NSLOT):
                    for dsc in idx_descs(c + 1, s):
                        dsc.wait()
                    gather(s, r).start()

                if b < 2:
                    prefetch()
                else:
                    pl.when(g < groups - 1)(prefetch)
                if b < 3:
                    launch()
                else:
                    pl.when(g < groups - 1)(launch)
                gather(b % ISLOT, b % NSLOT).wait()
                pltpu.sync_copy(rows.at[b % NSLOT], t_sh.at[didx.at[b % ISLOT]],
                                add=True)

        plsc.subcore_barrier()
        for j in range(rows_per_tile // zr):
            r = row0 + j * zr
            pltpu.sync_copy(t_sh.at[pl.ds(r, zr)],
                            out_hbm.at[pl.ds(cid * npad + r, zr)])

    return k(xs, src2d, dst2d).reshape(2, npad, d)


def _tc_finish(t2, xs, dinv_col, W, bias, prelu_w):
    n, d = xs.shape

    def body(t_ref, xs_ref, d_ref, w_ref, b_ref, p_ref, o_ref):
        v = (t_ref[0, :n] + t_ref[1, :n] + xs_ref[...]) * d_ref[...]
        z = jnp.dot(v, w_ref[...], preferred_element_type=jnp.float32)
        z = z + b_ref[...][None, :]
        o_ref[...] = jnp.where(z >= 0, z, p_ref[...][None, :] * z)

    return pl.pallas_call(
        body, out_shape=jax.ShapeDtypeStruct((n, W.shape[1]), jnp.float32)
    )(t2, xs, dinv_col, W, bias, prelu_w)


def kernel(x, edge_index, W, bias, prelu_w):
    n = x.shape[0]
    npad = -(-n // 2048) * 2048
    src = edge_index[0].astype(jnp.int32)
    dst = edge_index[1].astype(jnp.int32)

    e = src.shape[0]
    quant = N_WORKERS * CHUNK * NSLOT
    ep = -(-e // quant) * quant
    pad = ep - e
    src_p = jnp.concatenate([src, jnp.zeros((pad,), jnp.int32)])
    dst_p = jnp.concatenate([dst, jnp.full((pad,), npad - 1, jnp.int32)])

    hist = _sc_degree(dst_p, npad)
    dinv = _tc_dinv(hist)
    dinv_col = dinv[:n].reshape(n, 1)
    xs = _tc_scale(x, dinv_col)
    t2 = _sc_aggregate(xs, src_p.reshape(-1, CHUNK), dst_p.reshape(-1, CHUNK),
                       npad)
    return _tc_finish(t2, xs, dinv_col, W, bias, prelu_w)


# speedup vs baseline: 1.1203x; 1.1203x over previous
import dataclasses
import functools

import jax
import jax.numpy as jnp
from jax import lax
from jax.experimental import pallas as pl
from jax.experimental.pallas import tpu as pltpu
from jax.experimental.pallas import tpu_sc as plsc

N_WORKERS = 32
LANES = 16
CHUNK = 128


def _sc_compiler_params():
    cp = pltpu.CompilerParams()
    if "needs_layout_passes" in pltpu.CompilerParams.__dataclass_fields__:
        cp = dataclasses.replace(cp, needs_layout_passes=False)
    return cp


def _sc_degree(dst, n_bins):
    n_nodes = n_bins
    n_edges = dst.shape[0]
    per_w = n_edges // N_WORKERS
    assert per_w * N_WORKERS == n_edges and per_w % LANES == 0
    mesh = plsc.VectorSubcoreMesh(core_axis_name="c", subcore_axis_name="s")

    @functools.partial(
        pl.kernel,
        out_type=jax.ShapeDtypeStruct((N_WORKERS, n_nodes), jnp.float32),
        mesh=mesh,
        scratch_types=[
            pltpu.VMEM((per_w,), jnp.int32),
            pltpu.VMEM((n_nodes,), jnp.float32),
        ],
        compiler_params=_sc_compiler_params(),
    )
    def k(dst_hbm, out_hbm, idx_v, hist_v):
        wid = lax.axis_index("s") * 2 + lax.axis_index("c")
        zero16 = jnp.zeros((LANES,), jnp.float32)
        ones16 = jnp.ones((LANES,), jnp.float32)

        @pl.loop(0, n_nodes, step=LANES)
        def _(i):
            hist_v[pl.ds(i, LANES)] = zero16

        pltpu.sync_copy(dst_hbm.at[pl.ds(wid * per_w, per_w)], idx_v)

        @pl.loop(0, per_w, step=LANES)
        def _(i):
            idx = idx_v[pl.ds(i, LANES)]
            plsc.addupdate_scatter(hist_v, [idx], ones16)

        pltpu.sync_copy(hist_v, out_hbm.at[wid])

    return k(dst)


def _tc_dinv(hist):
    n = hist.shape[1]

    def body(h_ref, o_ref):
        deg = jnp.sum(h_ref[...], axis=0) + 1.0
        o_ref[...] = lax.rsqrt(deg)

    return pl.pallas_call(
        body, out_shape=jax.ShapeDtypeStruct((n,), jnp.float32)
    )(hist)


def _tc_scale(x, dinv_col):
    def body(x_ref, d_ref, o_ref):
        o_ref[...] = x_ref[...] * d_ref[...]

    return pl.pallas_call(
        body, out_shape=jax.ShapeDtypeStruct(x.shape, x.dtype)
    )(x, dinv_col)


NSLOT = 2
ISLOT = 4


def _sc_aggregate(xs, src2d, dst2d, npad):
    n, d = xs.shape
    total_chunks = src2d.shape[0]
    cpt = total_chunks // N_WORKERS
    assert cpt * N_WORKERS == total_chunks and cpt % ISLOT == 0
    groups = cpt // ISLOT
    rows_per_tile = npad // 16
    zr = 32
    assert rows_per_tile % zr == 0
    mesh = plsc.VectorSubcoreMesh(core_axis_name="c", subcore_axis_name="s")

    @functools.partial(
        pl.kernel,
        out_type=jax.ShapeDtypeStruct((2 * npad, d), jnp.float32),
        mesh=mesh,
        scratch_types=[
            pltpu.VMEM_SHARED((npad, d), jnp.float32),
            pltpu.VMEM((zr, d), jnp.float32),
            pltpu.VMEM((ISLOT, CHUNK), jnp.int32),
            pltpu.VMEM((ISLOT, CHUNK), jnp.int32),
            pltpu.VMEM((NSLOT, CHUNK, d), jnp.float32),
            pltpu.SemaphoreType.DMA((ISLOT,)),
            pltpu.SemaphoreType.DMA((NSLOT,)),
        ],
    )
    def k(xs_hbm, src_hbm, dst_hbm, out_hbm, t_sh, zbuf, sidx, didx, rows,
          isem, gsem):
        cid = lax.axis_index("c")
        sid = lax.axis_index("s")
        wid = cid * 16 + sid
        crow0 = wid * cpt
        zero16 = jnp.zeros((LANES,), jnp.float32)

        @pl.loop(0, zr)
        def _(r):
            for j in range(d // LANES):
                zbuf[r, pl.ds(j * LANES, LANES)] = zero16

        row0 = sid * rows_per_tile
        for j in range(rows_per_tile // zr):
            pltpu.sync_copy(zbuf, t_sh.at[pl.ds(row0 + j * zr, zr)])
        plsc.subcore_barrier()

        def idx_descs(c, s):
            return (
                pltpu.make_async_copy(src_hbm.at[crow0 + c], sidx.at[s],
                                      isem.at[s]),
                pltpu.make_async_copy(dst_hbm.at[crow0 + c], didx.at[s],
                                      isem.at[s]),
            )

        def gather(s, b):
            return pltpu.make_async_copy(
                xs_hbm.at[sidx.at[s]], rows.at[b], gsem.at[b])

        for c0 in range(2):
            for dsc in idx_descs(c0, c0):
                dsc.start()
        for dsc in idx_descs(0, 0):
            dsc.wait()
        gather(0, 0).start()

        @pl.loop(0, groups)
        def _(g):
            for b in range(ISLOT):
                c = g * ISLOT + b

                def prefetch(c=c, s=(b + 2) % ISLOT):
                    for dsc in idx_descs(c + 2, s):
                        dsc.start()

                def launch(c=c, s=(b + 1) % ISLOT, r=(b + 1) % NSLOT):
                    for dsc in idx_descs(c + 1, s):
                        dsc.wait()
                    gather(s, r).start()

                if b < 2:
                    prefetch()
                else:
                    pl.when(g < groups - 1)(prefetch)
                if b < 3:
                    launch()
                else:
                    pl.when(g < groups - 1)(launch)
                gather(b % ISLOT, b % NSLOT).wait()
                pltpu.sync_copy(rows.at[b % NSLOT], t_sh.at[didx.at[b % ISLOT]],
                                add=True)

        plsc.subcore_barrier()
        for j in range(rows_per_tile // zr):
            r = row0 + j * zr
            pltpu.sync_copy(t_sh.at[pl.ds(r, zr)],
                            out_hbm.at[pl.ds(cid * npad + r, zr)])

    return k(xs, src2d, dst2d).reshape(2, npad, d)


def _tc_finish(t2, xs, dinv_col, W, bias, prelu_w):
    n, d = xs.shape

    def body(t_ref, xs_ref, d_ref, w_ref, b_ref, p_ref, o_ref):
        v = (t_ref[0, :n] + t_ref[1, :n] + xs_ref[...]) * d_ref[...]
        z = jnp.dot(v, w_ref[...], preferred_element_type=jnp.float32)
        z = z + b_ref[...][None, :]
        o_ref[...] = jnp.where(z >= 0, z, p_ref[...][None, :] * z)

    return pl.pallas_call(
        body, out_shape=jax.ShapeDtypeStruct((n, W.shape[1]), jnp.float32)
    )(t2, xs, dinv_col, W, bias, prelu_w)


def kernel(x, edge_index, W, bias, prelu_w):
    n = x.shape[0]
    npad = -(-n // 2048) * 2048
    src = edge_index[0].astype(jnp.int32)
    dst = edge_index[1].astype(jnp.int32)

    e = src.shape[0]
    quant = N_WORKERS * CHUNK * NSLOT
    ep = -(-e // quant) * quant
    pad = ep - e
    src_p = jnp.concatenate([src, jnp.zeros((pad,), jnp.int32)])
    dst_p = jnp.concatenate(
        [dst, n + jnp.arange(pad, dtype=jnp.int32) % (npad - n)])

    hist = _sc_degree(dst_p, npad)
    dinv = _tc_dinv(hist)
    dinv_col = dinv[:n].reshape(n, 1)
    xs = _tc_scale(x, dinv_col)
    t2 = _sc_aggregate(xs, src_p.reshape(-1, CHUNK), dst_p.reshape(-1, CHUNK),
                       npad)
    return _tc_finish(t2, xs, dinv_col, W, bias, prelu_w)


# speedup vs baseline: 3.4719x; 3.0992x over previous
import dataclasses
import functools

import jax
import jax.numpy as jnp
from jax import lax
from jax.experimental import pallas as pl
from jax.experimental.pallas import tpu as pltpu
from jax.experimental.pallas import tpu_sc as plsc

N_WORKERS = 32
LANES = 16
CHUNK = 128


def _sc_compiler_params():
    cp = pltpu.CompilerParams()
    if "needs_layout_passes" in pltpu.CompilerParams.__dataclass_fields__:
        cp = dataclasses.replace(cp, needs_layout_passes=False)
    return cp


def _sc_degree(dst, n_bins):
    n_nodes = n_bins
    n_edges = dst.shape[0]
    per_w = n_edges // N_WORKERS
    assert per_w * N_WORKERS == n_edges and per_w % LANES == 0
    mesh = plsc.VectorSubcoreMesh(core_axis_name="c", subcore_axis_name="s")

    @functools.partial(
        pl.kernel,
        out_type=jax.ShapeDtypeStruct((N_WORKERS, n_nodes), jnp.float32),
        mesh=mesh,
        scratch_types=[
            pltpu.VMEM((per_w,), jnp.int32),
            pltpu.VMEM((n_nodes,), jnp.float32),
        ],
        compiler_params=_sc_compiler_params(),
    )
    def k(dst_hbm, out_hbm, idx_v, hist_v):
        wid = lax.axis_index("s") * 2 + lax.axis_index("c")
        zero16 = jnp.zeros((LANES,), jnp.float32)
        ones16 = jnp.ones((LANES,), jnp.float32)

        @pl.loop(0, n_nodes, step=LANES)
        def _(i):
            hist_v[pl.ds(i, LANES)] = zero16

        pltpu.sync_copy(dst_hbm.at[pl.ds(wid * per_w, per_w)], idx_v)

        @pl.loop(0, per_w, step=LANES)
        def _(i):
            idx = idx_v[pl.ds(i, LANES)]
            plsc.addupdate_scatter(hist_v, [idx], ones16)

        pltpu.sync_copy(hist_v, out_hbm.at[wid])

    return k(dst)


def _tc_dinv(hist):
    n = hist.shape[1]

    def body(h_ref, o_ref):
        deg = jnp.sum(h_ref[...], axis=0) + 1.0
        o_ref[...] = lax.rsqrt(deg)

    return pl.pallas_call(
        body, out_shape=jax.ShapeDtypeStruct((n,), jnp.float32)
    )(hist)


def _tc_scale(x, dinv_col):
    def body(x_ref, d_ref, o_ref):
        o_ref[...] = x_ref[...] * d_ref[...]

    return pl.pallas_call(
        body, out_shape=jax.ShapeDtypeStruct(x.shape, x.dtype)
    )(x, dinv_col)


NSLOT = 2
ISLOT = 4


def _sc_aggregate(xs, src2d, dst2d, npad):
    n, d = xs.shape
    total_chunks = src2d.shape[0]
    cpt = total_chunks // N_WORKERS
    assert cpt * N_WORKERS == total_chunks and cpt % ISLOT == 0
    groups = cpt // ISLOT
    rows_per_tile = npad // 16
    zr = 32
    assert rows_per_tile % zr == 0
    mesh = plsc.VectorSubcoreMesh(core_axis_name="c", subcore_axis_name="s")

    @functools.partial(
        pl.kernel,
        out_type=jax.ShapeDtypeStruct((2 * npad, d), jnp.float32),
        mesh=mesh,
        scratch_types=[
            pltpu.VMEM_SHARED((npad, d), jnp.float32),
            pltpu.VMEM((zr, d), jnp.float32),
            pltpu.VMEM((ISLOT, CHUNK), jnp.int32),
            pltpu.VMEM((ISLOT, CHUNK), jnp.int32),
            pltpu.VMEM((NSLOT, CHUNK, d), jnp.float32),
            pltpu.SemaphoreType.DMA((ISLOT,)),
            pltpu.SemaphoreType.DMA((NSLOT,)),
        ],
    )
    def k(xs_hbm, src_hbm, dst_hbm, out_hbm, t_sh, zbuf, sidx, didx, rows,
          isem, gsem):
        cid = lax.axis_index("c")
        sid = lax.axis_index("s")
        wid = cid * 16 + sid
        crow0 = wid * cpt
        zero16 = jnp.zeros((LANES,), jnp.float32)

        @pl.loop(0, zr)
        def _(r):
            for j in range(d // LANES):
                zbuf[r, pl.ds(j * LANES, LANES)] = zero16

        row0 = sid * rows_per_tile
        for j in range(rows_per_tile // zr):
            pltpu.sync_copy(zbuf, t_sh.at[pl.ds(row0 + j * zr, zr)])
        plsc.subcore_barrier()

        def idx_descs(c, s):
            return (
                pltpu.make_async_copy(src_hbm.at[crow0 + c], sidx.at[s],
                                      isem.at[s]),
                pltpu.make_async_copy(dst_hbm.at[crow0 + c], didx.at[s],
                                      isem.at[s]),
            )

        def gather(s, b):
            return pltpu.make_async_copy(
                xs_hbm.at[sidx.at[s]], rows.at[b], gsem.at[b])

        for c0 in range(2):
            for dsc in idx_descs(c0, c0):
                dsc.start()
        for dsc in idx_descs(0, 0):
            dsc.wait()
        gather(0, 0).start()

        @pl.loop(0, groups)
        def _(g):
            for b in range(ISLOT):
                c = g * ISLOT + b

                def prefetch(c=c, s=(b + 2) % ISLOT):
                    for dsc in idx_descs(c + 2, s):
                        dsc.start()

                def launch(c=c, s=(b + 1) % ISLOT, r=(b + 1) % NSLOT):
                    for dsc in idx_descs(c + 1, s):
                        dsc.wait()
                    gather(s, r).start()

                if b < 2:
                    prefetch()
                else:
                    pl.when(g < groups - 1)(prefetch)
                if b < 3:
                    launch()
                else:
                    pl.when(g < groups - 1)(launch)
                gather(b % ISLOT, b % NSLOT).wait()
                pltpu.sync_copy(rows.at[b % NSLOT], t_sh.at[didx.at[b % ISLOT]],
                                add=True)

        plsc.subcore_barrier()
        for j in range(rows_per_tile // zr):
            r = row0 + j * zr
            pltpu.sync_copy(t_sh.at[pl.ds(r, zr)],
                            out_hbm.at[pl.ds(cid * npad + r, zr)])

    return k(xs, src2d, dst2d).reshape(2, npad, d)


def _tc_finish(t2, xs, dinv_col, W, bias, prelu_w):
    n, d = xs.shape

    def body(t_ref, xs_ref, d_ref, w_ref, b_ref, p_ref, o_ref):
        v = (t_ref[0, :n] + t_ref[1, :n] + xs_ref[...]) * d_ref[...]
        z = jnp.dot(v, w_ref[...], preferred_element_type=jnp.float32)
        z = z + b_ref[...][None, :]
        o_ref[...] = jnp.where(z >= 0, z, p_ref[...][None, :] * z)

    return pl.pallas_call(
        body, out_shape=jax.ShapeDtypeStruct((n, W.shape[1]), jnp.float32)
    )(t2, xs, dinv_col, W, bias, prelu_w)


def kernel(x, edge_index, W, bias, prelu_w):
    n = x.shape[0]
    npad = -(-n // 2048) * 2048
    src = edge_index[0].astype(jnp.int32)
    dst = edge_index[1].astype(jnp.int32)

    e = src.shape[0]
    quant = N_WORKERS * CHUNK * NSLOT
    ep = -(-e // quant) * quant
    pad = ep - e
    src_p = jnp.concatenate([src, jnp.arange(pad, dtype=jnp.int32) % n])
    dst_p = jnp.concatenate(
        [dst, n + jnp.arange(pad, dtype=jnp.int32) % (npad - n)])

    hist = _sc_degree(dst_p, npad)
    dinv = _tc_dinv(hist)
    dinv_col = dinv[:n].reshape(n, 1)
    xs = _tc_scale(x, dinv_col)
    t2 = _sc_aggregate(xs, src_p.reshape(-1, CHUNK), dst_p.reshape(-1, CHUNK),
                       npad)
    return _tc_finish(t2, xs, dinv_col, W, bias, prelu_w)
